# Initial kernel scaffold; baseline (speedup 1.0000x reference)
import jax
import jax.numpy as jnp
from jax.experimental import pallas as pl


def kernel(x, gn1_scale, gn1_bias, conv_w, conv_b, gn2_scale, gn2_bias, router_w, w1, b1, w2, b2):
    raise NotImplementedError("write your pallas kernel here")


# speedup vs baseline: 1.9462x; 1.9462x over previous
import functools

import jax
import jax.numpy as jnp
from jax.experimental import pallas as pl
from jax.experimental.pallas import tpu as pltpu

NUM_EXPERTS = 8
C_IN = 96
C_HID = 384
GROUPS = 32
H = 64
W = 64
HW = H * W
PAD_OFF = 65
PAD_ROWS = 4232


def _gn(v, gm, scale, bias):
    n = (C_IN // GROUPS) * HW
    s = jnp.sum(v, axis=0, keepdims=True) @ gm
    ss = jnp.sum(v * v, axis=0, keepdims=True) @ gm
    mean = s / n
    var = ss / n - mean * mean
    return (v - mean) * jax.lax.rsqrt(var + 1e-6) * scale + bias


def _gelu_tanh(x):
    return 0.5 * x * (1.0 + jnp.tanh(0.7978845608028654 * (x + 0.044715 * x * x * x)))


def _block_kernel(x_ref, gm_ref, g1s_ref, g1b_ref, wt_ref, cb_ref, g2s_ref,
                  g2b_ref, rw_ref, w1_ref, b1_ref, w2_ref, b2_ref,
                  out_ref, pad_ref):
    xb = x_ref[0]
    f32 = jnp.float32
    gm = gm_ref[...]

    h = _gn(xb, gm, g1s_ref[...], g1b_ref[...])
    h = h * jax.nn.sigmoid(h)

    pad_ref[...] = jnp.zeros((PAD_ROWS, C_IN), f32)
    pad_ref[pl.ds(PAD_OFF, HW), :] = h

    col = jax.lax.broadcasted_iota(jnp.int32, (HW, 1), 0) % W
    acc = jnp.zeros((HW, C_IN), f32) + cb_ref[...]
    for ky in range(3):
        for kx in range(3):
            dy, dx = ky - 1, kx - 1
            sl = pad_ref[pl.ds(PAD_OFF + dy * W + dx, HW), :]
            if dx == 1:
                sl = jnp.where(col == W - 1, 0.0, sl)
            elif dx == -1:
                sl = jnp.where(col == 0, 0.0, sl)
            acc += jnp.dot(sl, wt_ref[ky * 3 + kx], preferred_element_type=f32)

    t = _gn(acc, gm, g2s_ref[...], g2b_ref[...])
    t = t * jax.nn.sigmoid(t)

    logits = jnp.dot(t, rw_ref[...], preferred_element_type=f32)
    mx = jnp.max(logits, axis=1, keepdims=True)
    ex = jnp.exp(logits - mx)
    probs = ex / jnp.sum(ex, axis=1, keepdims=True)
    eidx = jax.lax.broadcasted_iota(jnp.int32, (HW, NUM_EXPERTS), 1)
    m1 = jnp.max(probs, axis=1, keepdims=True)
    i1 = jnp.min(jnp.where(probs == m1, eidx, NUM_EXPERTS), axis=1, keepdims=True)
    masked = jnp.where(eidx == i1, -1.0, probs)
    m2 = jnp.max(masked, axis=1, keepdims=True)
    i2 = jnp.min(jnp.where(masked == m2, eidx, NUM_EXPERTS), axis=1, keepdims=True)
    gates = (jnp.where(eidx == i1, m1, 0.0) + jnp.where(eidx == i2, m2, 0.0)) / (m1 + m2)

    moe = jnp.zeros((HW, C_IN), f32)
    for e in range(NUM_EXPERTS):
        h1 = jnp.dot(t, w1_ref[e], preferred_element_type=f32) + b1_ref[e]
        h1 = _gelu_tanh(h1)
        h2 = jnp.dot(h1, w2_ref[e], preferred_element_type=f32) + b2_ref[e]
        moe += gates[:, e:e + 1] * h2

    out_ref[0] = xb + moe


@jax.jit
def kernel(x, gn1_scale, gn1_bias, conv_w, conv_b, gn2_scale, gn2_bias,
           router_w, w1, b1, w2, b2):
    B = x.shape[0]
    xt = x.transpose(0, 2, 3, 1).reshape(B, HW, C_IN)
    gm = jnp.kron(jnp.eye(GROUPS, dtype=jnp.float32),
                  jnp.ones((C_IN // GROUPS, C_IN // GROUPS), jnp.float32))
    wt = conv_w.transpose(2, 3, 1, 0).reshape(9, C_IN, C_IN)

    const = lambda *shape: pl.BlockSpec(shape, lambda b: (0,) * len(shape))
    out = pl.pallas_call(
        _block_kernel,
        grid=(B,),
        in_specs=[
            pl.BlockSpec((1, HW, C_IN), lambda b: (b, 0, 0)),
            const(C_IN, C_IN),
            const(1, C_IN), const(1, C_IN),
            const(9, C_IN, C_IN), const(1, C_IN),
            const(1, C_IN), const(1, C_IN),
            const(C_IN, NUM_EXPERTS),
            const(NUM_EXPERTS, C_IN, C_HID), const(NUM_EXPERTS, 1, C_HID),
            const(NUM_EXPERTS, C_HID, C_IN), const(NUM_EXPERTS, 1, C_IN),
        ],
        out_specs=pl.BlockSpec((1, HW, C_IN), lambda b: (b, 0, 0)),
        out_shape=jax.ShapeDtypeStruct((B, HW, C_IN), jnp.float32),
        scratch_shapes=[pltpu.VMEM((PAD_ROWS, C_IN), jnp.float32)],
        compiler_params=pltpu.CompilerParams(
            dimension_semantics=("parallel",)),
    )(xt, gm, gn1_scale.reshape(1, C_IN), gn1_bias.reshape(1, C_IN), wt,
      conv_b.reshape(1, C_IN), gn2_scale.reshape(1, C_IN),
      gn2_bias.reshape(1, C_IN), router_w,
      w1, b1.reshape(NUM_EXPERTS, 1, C_HID),
      w2, b2.reshape(NUM_EXPERTS, 1, C_IN))

    return out.reshape(B, H, W, C_IN).transpose(0, 3, 1, 2)
